# Initial kernel scaffold; baseline (speedup 1.0000x reference)
import jax
import jax.numpy as jnp
from jax.experimental import pallas as pl


def kernel(inputs, W):
    raise NotImplementedError("write your pallas kernel here")


# speedup vs baseline: 2.9066x; 2.9066x over previous
import jax
import jax.numpy as jnp
from jax.experimental import pallas as pl
from jax.experimental.pallas import tpu as pltpu

N_E = 1024
D = 64
N_ROWS = 16 * 32 * 32
BLK = 256
N_BLKS = N_ROWS // BLK
COMMIT = 0.25


def _vq_block_kernel(flat_ref, w_ref, enc_ref, idx_ref, q_ref, loss_ref,
                     ppl_ref, sse_acc, cnt_acc):
    i = pl.program_id(0)
    flat = flat_ref[...]
    w = w_ref[...]

    a = jnp.sum(flat * flat, axis=1, keepdims=True)
    b = jnp.sum(w * w, axis=1)
    scores = jax.lax.dot_general(
        flat, w, (((1,), (1,)), ((), ())),
        preferred_element_type=jnp.float32)
    dist = (a + b[None, :]) - 2.0 * scores

    mval = jnp.min(dist, axis=1, keepdims=True)
    iota = jax.lax.broadcasted_iota(jnp.int32, (BLK, N_E), 1)
    idx = jnp.min(jnp.where(dist == mval, iota, N_E), axis=1)
    idx_ref[...] = idx[:, None]

    onehot = (iota == idx[:, None]).astype(jnp.float32)
    enc_ref[...] = onehot

    q = jax.lax.dot_general(
        onehot, w, (((1,), (0,)), ((), ())),
        preferred_element_type=jnp.float32)
    q_ref[...] = flat + (q - flat)

    diff = q - flat
    part = jnp.sum(diff * diff, keepdims=True)
    cnt = jnp.sum(onehot, axis=0, keepdims=True)

    @pl.when(i == 0)
    def _init():
        sse_acc[...] = part
        cnt_acc[...] = cnt

    @pl.when(i > 0)
    def _accum():
        sse_acc[...] += part
        cnt_acc[...] += cnt

    @pl.when(i == N_BLKS - 1)
    def _finalize():
        m = sse_acc[...] * (1.0 / float(N_ROWS * D))
        loss_ref[...] = m + COMMIT * m
        avg = cnt_acc[...] * (1.0 / float(N_ROWS))
        ent = jnp.sum(avg * jnp.log(avg + 1e-10), axis=1, keepdims=True)
        ppl_ref[...] = jnp.exp(-ent)


def kernel(inputs, W):
    z = jnp.transpose(inputs, (0, 2, 3, 1))
    flat = z.reshape(N_ROWS, D)

    enc, idx, q, loss, ppl = pl.pallas_call(
        _vq_block_kernel,
        grid=(N_BLKS,),
        in_specs=[
            pl.BlockSpec((BLK, D), lambda i: (i, 0)),
            pl.BlockSpec((N_E, D), lambda i: (0, 0)),
        ],
        out_specs=[
            pl.BlockSpec((BLK, N_E), lambda i: (i, 0)),
            pl.BlockSpec((BLK, 1), lambda i: (i, 0)),
            pl.BlockSpec((BLK, D), lambda i: (i, 0)),
            pl.BlockSpec((1, 1), lambda i: (0, 0)),
            pl.BlockSpec((1, 1), lambda i: (0, 0)),
        ],
        out_shape=[
            jax.ShapeDtypeStruct((N_ROWS, N_E), jnp.float32),
            jax.ShapeDtypeStruct((N_ROWS, 1), jnp.int32),
            jax.ShapeDtypeStruct((N_ROWS, D), jnp.float32),
            jax.ShapeDtypeStruct((1, 1), jnp.float32),
            jax.ShapeDtypeStruct((1, 1), jnp.float32),
        ],
        scratch_shapes=[
            pltpu.VMEM((1, 1), jnp.float32),
            pltpu.VMEM((1, N_E), jnp.float32),
        ],
    )(flat, W)

    quantized_out = jnp.transpose(q.reshape(16, 32, 32, D), (0, 3, 1, 2))
    return (loss[0, 0], quantized_out, ppl[0, 0], enc, idx)
